# baseline (speedup 1.0000x reference)
import functools

import jax
import jax.numpy as jnp
from jax import lax
from jax.experimental import pallas as pl
from jax.experimental.pallas import tpu as pltpu
from jax.experimental.pallas import tpu_sc as plsc

N = 10000
E = 160000
H = 512
NC, NS = 2, 16
NW = NC * NS
EB = 128
NB = 40
PW = NB * EB
E_PAD = NW * PW
CH = 128
NK = H // CH
NP = 10240
RPT = NP // NS
NBUF = 2

_sc_mesh = plsc.VectorSubcoreMesh(
    core_axis_name="c", subcore_axis_name="s", num_cores=NC, num_subcores=NS)


def _conv_body(with_counts, *refs):
    tables, ones, src2d, dst2d, zeros, out = refs[:6]
    srcs_v, dsts_v = refs[6:8]
    rows = refs[8:8 + NBUF]
    sems = refs[8 + NBUF:8 + 2 * NBUF]
    accum = refs[8 + 2 * NBUF]

    cid = lax.axis_index("c")
    sid = lax.axis_index("s")
    wid = cid * NS + sid
    stripe = pl.ds(sid * RPT, RPT)

    pltpu.sync_copy(src2d.at[pl.ds(wid * NB, NB)], srcs_v)
    pltpu.sync_copy(dst2d.at[pl.ds(wid * NB, NB)], dsts_v)

    def _zero_stripe():
        pltpu.sync_copy(zeros, rows[0])
        for t in range(RPT // EB):
            pltpu.sync_copy(rows[0],
                            accum.at[pl.ds(sid * RPT + t * EB, EB)])

    for c in range(NK):
        table = tables.at[c]
        _zero_stripe()
        plsc.subcore_barrier()

        for b in range(NBUF):
            pltpu.async_copy(table.at[srcs_v.at[b]], rows[b], sems[b])

        def group(g, carry, table=table):
            for b in range(NBUF):
                j = g * NBUF + b
                pltpu.make_async_copy(table.at[pl.ds(0, EB)],
                                      rows[b], sems[b]).wait()
                pltpu.sync_copy(rows[b], accum.at[dsts_v.at[j]], add=True)

                @pl.when(j + NBUF < NB)
                def _prefetch(b=b, j=j, table=table):
                    pltpu.async_copy(table.at[srcs_v.at[j + NBUF]],
                                     rows[b], sems[b])
            return carry

        lax.fori_loop(0, NB // NBUF, group, 0)
        plsc.subcore_barrier()
        pltpu.sync_copy(accum.at[stripe], out.at[c, cid, stripe])

    if with_counts:
        _zero_stripe()
        plsc.subcore_barrier()
        pltpu.sync_copy(ones, rows[0])

        def cgroup(j, carry):
            pltpu.sync_copy(rows[0], accum.at[dsts_v.at[j]], add=True)
            return carry

        lax.fori_loop(0, NB, cgroup, 0)
        plsc.subcore_barrier()
        pltpu.sync_copy(accum.at[stripe], out.at[NK, cid, stripe])


def _make_conv(with_counts):
    return pl.kernel(
        functools.partial(_conv_body, with_counts),
        out_type=jax.ShapeDtypeStruct(
            (NK + (1 if with_counts else 0), NC, NP, CH), jnp.float32),
        mesh=_sc_mesh,
        scratch_types=(
            [pltpu.VMEM((NB, EB), jnp.int32)] * 2
            + [pltpu.VMEM((EB, CH), jnp.float32)] * NBUF
            + [pltpu.SemaphoreType.DMA] * NBUF
            + [pltpu.VMEM_SHARED((NP, CH), jnp.float32)]
        ),
    )


_conv_sc = _make_conv(True)
_conv_sc_nc = _make_conv(False)


BM = 1000
_PREC = lax.Precision.HIGHEST


def _proj_body(x_ref, w_ref, b_ref, o_ref):
    acc = jnp.dot(x_ref[...], w_ref[...], precision=_PREC,
                  preferred_element_type=jnp.float32)
    acc = jnp.maximum(acc + b_ref[...], 0.0)
    for k in range(NK):
        o_ref[k] = acc[:, k * CH:(k + 1) * CH]


def _proj(x, w, b):
    d = x.shape[1]
    return pl.pallas_call(
        _proj_body,
        grid=(N // BM,),
        in_specs=[
            pl.BlockSpec((BM, d), lambda i: (i, 0)),
            pl.BlockSpec((d, H), lambda i: (0, 0)),
            pl.BlockSpec((1, H), lambda i: (0, 0)),
        ],
        out_specs=pl.BlockSpec((NK, BM, CH), lambda i: (0, i, 0)),
        out_shape=jax.ShapeDtypeStruct((NK, N, CH), jnp.float32),
    )(x, w, b.reshape(1, H))


def _mean_matmul(parts, cnt, x_ref, wl, wr, bl):
    c = cnt[0, 0] + cnt[0, 1]
    inv = 1.0 / jnp.maximum(c[:, :1], 1.0)
    acc = bl[...] + jnp.zeros((x_ref.shape[1], wl.shape[1]), jnp.float32)
    for k in range(NK):
        m = (parts[k, 0] + parts[k, 1]) * inv
        acc += jnp.dot(m, wl[k * CH:(k + 1) * CH, :], precision=_PREC,
                       preferred_element_type=jnp.float32)
        acc += jnp.dot(x_ref[k], wr[k * CH:(k + 1) * CH, :], precision=_PREC,
                       preferred_element_type=jnp.float32)
    return acc


def _sage_body(relu, chunked, p_ref, cnt_ref, x_ref, wl, bl, wr, o_ref):
    acc = _mean_matmul(p_ref, cnt_ref, x_ref, wl, wr, bl)
    acc = jnp.maximum(acc, 0.0) if relu else acc
    if chunked:
        for k in range(NK):
            o_ref[k] = acc[:, k * CH:(k + 1) * CH]
    else:
        o_ref[...] = acc


def _sage_cls_body(p_ref, cnt_ref, x_ref, wl, bl, wr, wc, bc, o_ref, lg_ref):
    acc = _mean_matmul(p_ref, cnt_ref, x_ref, wl, wr, bl)
    o_ref[...] = acc
    lg_ref[...] = jnp.dot(acc, wc[...], precision=_PREC,
                          preferred_element_type=jnp.float32) + bc[...]


_P_SPEC = pl.BlockSpec((NK, NC, BM, CH), lambda i: (0, 0, i, 0))
_CNT_SPEC = pl.BlockSpec((1, NC, BM, CH), lambda i: (NK, 0, i, 0))
_X_SPEC = pl.BlockSpec((NK, BM, CH), lambda i: (0, i, 0))
_WL_SPEC = pl.BlockSpec((H, H), lambda i: (0, 0))
_B_SPEC = pl.BlockSpec((1, H), lambda i: (0, 0))


def _sage_layer(parts, cnt, x, wl, bl, wr, relu, chunked):
    if chunked:
        out_spec = pl.BlockSpec((NK, BM, CH), lambda i: (0, i, 0))
        out_shape = jax.ShapeDtypeStruct((NK, N, CH), jnp.float32)
    else:
        out_spec = pl.BlockSpec((BM, H), lambda i: (i, 0))
        out_shape = jax.ShapeDtypeStruct((N, H), jnp.float32)
    return pl.pallas_call(
        functools.partial(_sage_body, relu, chunked),
        grid=(N // BM,),
        in_specs=[_P_SPEC, _CNT_SPEC, _X_SPEC, _WL_SPEC, _B_SPEC, _WL_SPEC],
        out_specs=out_spec,
        out_shape=out_shape,
    )(parts, cnt, x, wl, bl.reshape(1, H), wr)


def _sage_layer_cls(parts, cnt, x, wl, bl, wr, wc, bc):
    return pl.pallas_call(
        _sage_cls_body,
        grid=(N // BM,),
        in_specs=[
            _P_SPEC, _CNT_SPEC, _X_SPEC, _WL_SPEC, _B_SPEC, _WL_SPEC,
            pl.BlockSpec((H, 128), lambda i: (0, 0)),
            pl.BlockSpec((1, 128), lambda i: (0, 0)),
        ],
        out_specs=[pl.BlockSpec((BM, H), lambda i: (i, 0)),
                   pl.BlockSpec((BM, 128), lambda i: (i, 0))],
        out_shape=[jax.ShapeDtypeStruct((N, H), jnp.float32),
                   jax.ShapeDtypeStruct((N, 128), jnp.float32)],
    )(parts, cnt, x, wl, bl.reshape(1, H), wr, wc, bc)


def kernel(x_card, x_user, edge_index_user_card, edge_index_card_user,
           W_in_card, b_in_card, W_in_user, b_in_user,
           Wl_u2c_0, bl_u2c_0, Wr_u2c_0, Wl_c2u_0, bl_c2u_0, Wr_c2u_0,
           Wl_u2c_1, bl_u2c_1, Wr_u2c_1, Wl_c2u_1, bl_c2u_1, Wr_c2u_1,
           W_cls, b_cls):
    pad0 = jnp.zeros((E_PAD - E,), jnp.int32)
    padN = jnp.full((E_PAD - E,), N, jnp.int32)
    src_uc = jnp.concatenate([edge_index_user_card[0], pad0]).reshape(-1, EB)
    dst_uc = jnp.concatenate([edge_index_user_card[1], padN]).reshape(-1, EB)
    src_cu = jnp.concatenate([edge_index_card_user[0], pad0]).reshape(-1, EB)
    dst_cu = jnp.concatenate([edge_index_card_user[1], padN]).reshape(-1, EB)

    zeros = jnp.zeros((EB, CH), jnp.float32)
    ones = jnp.ones((EB, CH), jnp.float32)

    h_card = _proj(x_card, W_in_card, b_in_card)
    h_user = _proj(x_user, W_in_user, b_in_user)

    pc0 = _conv_sc(h_user, ones, src_uc, dst_uc, zeros)
    pu0 = _conv_sc(h_card, ones, src_cu, dst_cu, zeros)
    h_card1 = _sage_layer(pc0, pc0, h_card, Wl_u2c_0, bl_u2c_0,
                          Wr_u2c_0, relu=True, chunked=True)
    h_user1 = _sage_layer(pu0, pu0, h_user, Wl_c2u_0, bl_c2u_0,
                          Wr_c2u_0, relu=True, chunked=True)

    pc1 = _conv_sc_nc(h_user1, ones, src_uc, dst_uc, zeros)
    pu1 = _conv_sc_nc(h_card1, ones, src_cu, dst_cu, zeros)
    wc_pad = jnp.zeros((H, 128), jnp.float32).at[:, :2].set(W_cls)
    bc_pad = jnp.zeros((1, 128), jnp.float32).at[0, :2].set(b_cls)
    h_card2, logits_pad = _sage_layer_cls(pc1, pc0, h_card1, Wl_u2c_1,
                                          bl_u2c_1, Wr_u2c_1, wc_pad, bc_pad)
    h_user2 = _sage_layer(pu1, pu0, h_user1, Wl_c2u_1, bl_c2u_1,
                          Wr_c2u_1, relu=False, chunked=False)

    return logits_pad[:, :2], h_card2, h_user2


# speedup vs baseline: 1.0535x; 1.0535x over previous
import functools

import jax
import jax.numpy as jnp
from jax import lax
from jax.experimental import pallas as pl
from jax.experimental.pallas import tpu as pltpu
from jax.experimental.pallas import tpu_sc as plsc

N = 10000
E = 160000
H = 512
NC, NS = 2, 16
NW = NC * NS
EB = 64
NB = 80
PW = NB * EB
E_PAD = NW * PW
CH = 128
NK = H // CH
NP = 10240
RPT = NP // NS
NBUF = 4

_sc_mesh = plsc.VectorSubcoreMesh(
    core_axis_name="c", subcore_axis_name="s", num_cores=NC, num_subcores=NS)


def _conv_body(with_counts, *refs):
    tables, ones, src2d, dst2d, zeros, out = refs[:6]
    srcs_v, dsts_v = refs[6:8]
    rows = refs[8:8 + NBUF]
    sems = refs[8 + NBUF:8 + 2 * NBUF]
    accum = refs[8 + 2 * NBUF]

    cid = lax.axis_index("c")
    sid = lax.axis_index("s")
    wid = cid * NS + sid
    stripe = pl.ds(sid * RPT, RPT)

    pltpu.sync_copy(src2d.at[pl.ds(wid * (NB // 2), NB // 2)], srcs_v)
    pltpu.sync_copy(dst2d.at[pl.ds(wid * NB, NB)], dsts_v)

    def _src_slice(j):
        if isinstance(j, int):
            return srcs_v.at[j // 2, pl.ds((j % 2) * EB, EB)]
        return srcs_v.at[j >> 1, pl.ds((j & 1) * EB, EB)]

    def _zero_stripe():
        pltpu.sync_copy(zeros, rows[0])
        for t in range(RPT // EB):
            pltpu.sync_copy(rows[0],
                            accum.at[pl.ds(sid * RPT + t * EB, EB)])

    for c in range(NK):
        table = tables.at[c]
        _zero_stripe()
        plsc.subcore_barrier()

        for b in range(NBUF):
            pltpu.async_copy(table.at[_src_slice(b)], rows[b], sems[b])

        def group(g, carry, table=table):
            for b in range(NBUF):
                j = g * NBUF + b
                pltpu.make_async_copy(table.at[pl.ds(0, EB)],
                                      rows[b], sems[b]).wait()
                pltpu.sync_copy(rows[b], accum.at[dsts_v.at[j]], add=True)

                @pl.when(j + NBUF < NB)
                def _prefetch(b=b, j=j, table=table):
                    pltpu.async_copy(table.at[_src_slice(j + NBUF)],
                                     rows[b], sems[b])
            return carry

        lax.fori_loop(0, NB // NBUF, group, 0)
        plsc.subcore_barrier()
        pltpu.sync_copy(accum.at[stripe], out.at[c, cid, stripe])

    if with_counts:
        _zero_stripe()
        plsc.subcore_barrier()
        pltpu.sync_copy(ones, rows[0])

        def cgroup(j, carry):
            pltpu.sync_copy(rows[0], accum.at[dsts_v.at[j]], add=True)
            return carry

        lax.fori_loop(0, NB, cgroup, 0)
        plsc.subcore_barrier()
        pltpu.sync_copy(accum.at[stripe], out.at[NK, cid, stripe])


def _make_conv(with_counts):
    return pl.kernel(
        functools.partial(_conv_body, with_counts),
        out_type=jax.ShapeDtypeStruct(
            (NK + (1 if with_counts else 0), NC, NP, CH), jnp.float32),
        mesh=_sc_mesh,
        scratch_types=(
            [pltpu.VMEM((NB // 2, 2 * EB), jnp.int32),
             pltpu.VMEM((NB, EB), jnp.int32)]
            + [pltpu.VMEM((EB, CH), jnp.float32)] * NBUF
            + [pltpu.SemaphoreType.DMA] * NBUF
            + [pltpu.VMEM_SHARED((NP, CH), jnp.float32)]
        ),
    )


_conv_sc = _make_conv(True)
_conv_sc_nc = _make_conv(False)


BM = 1000
_PREC = lax.Precision.HIGHEST


def _proj_body(x_ref, w_ref, b_ref, o_ref):
    acc = jnp.dot(x_ref[...], w_ref[...], precision=_PREC,
                  preferred_element_type=jnp.float32)
    acc = jnp.maximum(acc + b_ref[...], 0.0)
    for k in range(NK):
        o_ref[k] = acc[:, k * CH:(k + 1) * CH]


def _proj(x, w, b):
    d = x.shape[1]
    return pl.pallas_call(
        _proj_body,
        grid=(N // BM,),
        in_specs=[
            pl.BlockSpec((BM, d), lambda i: (i, 0)),
            pl.BlockSpec((d, H), lambda i: (0, 0)),
            pl.BlockSpec((1, H), lambda i: (0, 0)),
        ],
        out_specs=pl.BlockSpec((NK, BM, CH), lambda i: (0, i, 0)),
        out_shape=jax.ShapeDtypeStruct((NK, N, CH), jnp.float32),
    )(x, w, b.reshape(1, H))


def _mean_matmul(parts, cnt, x_ref, wl, wr, bl):
    c = cnt[0, 0] + cnt[0, 1]
    inv = 1.0 / jnp.maximum(c[:, :1], 1.0)
    acc = bl[...] + jnp.zeros((x_ref.shape[1], wl.shape[1]), jnp.float32)
    for k in range(NK):
        m = (parts[k, 0] + parts[k, 1]) * inv
        acc += jnp.dot(m, wl[k * CH:(k + 1) * CH, :], precision=_PREC,
                       preferred_element_type=jnp.float32)
        acc += jnp.dot(x_ref[k], wr[k * CH:(k + 1) * CH, :], precision=_PREC,
                       preferred_element_type=jnp.float32)
    return acc


def _sage_body(relu, chunked, p_ref, cnt_ref, x_ref, wl, bl, wr, o_ref):
    acc = _mean_matmul(p_ref, cnt_ref, x_ref, wl, wr, bl)
    acc = jnp.maximum(acc, 0.0) if relu else acc
    if chunked:
        for k in range(NK):
            o_ref[k] = acc[:, k * CH:(k + 1) * CH]
    else:
        o_ref[...] = acc


def _sage_cls_body(p_ref, cnt_ref, x_ref, wl, bl, wr, wc, bc, o_ref, lg_ref):
    acc = _mean_matmul(p_ref, cnt_ref, x_ref, wl, wr, bl)
    o_ref[...] = acc
    lg_ref[...] = jnp.dot(acc, wc[...], precision=_PREC,
                          preferred_element_type=jnp.float32) + bc[...]


_P_SPEC = pl.BlockSpec((NK, NC, BM, CH), lambda i: (0, 0, i, 0))
_CNT_SPEC = pl.BlockSpec((1, NC, BM, CH), lambda i: (NK, 0, i, 0))
_X_SPEC = pl.BlockSpec((NK, BM, CH), lambda i: (0, i, 0))
_WL_SPEC = pl.BlockSpec((H, H), lambda i: (0, 0))
_B_SPEC = pl.BlockSpec((1, H), lambda i: (0, 0))


def _sage_layer(parts, cnt, x, wl, bl, wr, relu, chunked):
    if chunked:
        out_spec = pl.BlockSpec((NK, BM, CH), lambda i: (0, i, 0))
        out_shape = jax.ShapeDtypeStruct((NK, N, CH), jnp.float32)
    else:
        out_spec = pl.BlockSpec((BM, H), lambda i: (i, 0))
        out_shape = jax.ShapeDtypeStruct((N, H), jnp.float32)
    return pl.pallas_call(
        functools.partial(_sage_body, relu, chunked),
        grid=(N // BM,),
        in_specs=[_P_SPEC, _CNT_SPEC, _X_SPEC, _WL_SPEC, _B_SPEC, _WL_SPEC],
        out_specs=out_spec,
        out_shape=out_shape,
    )(parts, cnt, x, wl, bl.reshape(1, H), wr)


def _sage_layer_cls(parts, cnt, x, wl, bl, wr, wc, bc):
    return pl.pallas_call(
        _sage_cls_body,
        grid=(N // BM,),
        in_specs=[
            _P_SPEC, _CNT_SPEC, _X_SPEC, _WL_SPEC, _B_SPEC, _WL_SPEC,
            pl.BlockSpec((H, 128), lambda i: (0, 0)),
            pl.BlockSpec((1, 128), lambda i: (0, 0)),
        ],
        out_specs=[pl.BlockSpec((BM, H), lambda i: (i, 0)),
                   pl.BlockSpec((BM, 128), lambda i: (i, 0))],
        out_shape=[jax.ShapeDtypeStruct((N, H), jnp.float32),
                   jax.ShapeDtypeStruct((N, 128), jnp.float32)],
    )(parts, cnt, x, wl, bl.reshape(1, H), wr, wc, bc)


def kernel(x_card, x_user, edge_index_user_card, edge_index_card_user,
           W_in_card, b_in_card, W_in_user, b_in_user,
           Wl_u2c_0, bl_u2c_0, Wr_u2c_0, Wl_c2u_0, bl_c2u_0, Wr_c2u_0,
           Wl_u2c_1, bl_u2c_1, Wr_u2c_1, Wl_c2u_1, bl_c2u_1, Wr_c2u_1,
           W_cls, b_cls):
    pad0 = jnp.zeros((E_PAD - E,), jnp.int32)
    padN = jnp.full((E_PAD - E,), N, jnp.int32)
    src_uc = jnp.concatenate([edge_index_user_card[0], pad0]).reshape(-1, 2 * EB)
    dst_uc = jnp.concatenate([edge_index_user_card[1], padN]).reshape(-1, EB)
    src_cu = jnp.concatenate([edge_index_card_user[0], pad0]).reshape(-1, 2 * EB)
    dst_cu = jnp.concatenate([edge_index_card_user[1], padN]).reshape(-1, EB)

    zeros = jnp.zeros((EB, CH), jnp.float32)
    ones = jnp.ones((EB, CH), jnp.float32)

    h_card = _proj(x_card, W_in_card, b_in_card)
    h_user = _proj(x_user, W_in_user, b_in_user)

    pc0 = _conv_sc(h_user, ones, src_uc, dst_uc, zeros)
    pu0 = _conv_sc(h_card, ones, src_cu, dst_cu, zeros)
    h_card1 = _sage_layer(pc0, pc0, h_card, Wl_u2c_0, bl_u2c_0,
                          Wr_u2c_0, relu=True, chunked=True)
    h_user1 = _sage_layer(pu0, pu0, h_user, Wl_c2u_0, bl_c2u_0,
                          Wr_c2u_0, relu=True, chunked=True)

    pc1 = _conv_sc_nc(h_user1, ones, src_uc, dst_uc, zeros)
    pu1 = _conv_sc_nc(h_card1, ones, src_cu, dst_cu, zeros)
    wc_pad = jnp.zeros((H, 128), jnp.float32).at[:, :2].set(W_cls)
    bc_pad = jnp.zeros((1, 128), jnp.float32).at[0, :2].set(b_cls)
    h_card2, logits_pad = _sage_layer_cls(pc1, pc0, h_card1, Wl_u2c_1,
                                          bl_u2c_1, Wr_u2c_1, wc_pad, bc_pad)
    h_user2 = _sage_layer(pu1, pu0, h_user1, Wl_c2u_1, bl_c2u_1,
                          Wr_c2u_1, relu=False, chunked=False)

    return logits_pad[:, :2], h_card2, h_user2
